# baseline (speedup 1.0000x reference)
import jax
import jax.numpy as jnp
from jax.experimental import pallas as pl
from jax.experimental.pallas import tpu as pltpu

B, J, V = 32, 8, 32768
VC = 2048
NCHUNK = V // VC


def _fused_body(*refs):
    a_refs = refs[:J]
    e_refs = refs[J:2 * J]
    logits_ref, stddev_ref, best_ref, m_ref, i_ref = refs[2 * J:]

    j = pl.program_id(0)

    a = [r[...] for r in a_refs]

    m = a[0]
    s = e_refs[0][...]
    for jj in range(1, J):
        upd = a[jj] < m
        m = jnp.minimum(m, a[jj])
        s = jnp.where(upd, e_refs[jj][...], s)
    logits_ref[...] = m
    stddev_ref[...] = s

    @pl.when(j == 0)
    def _():
        m_ref[...] = jnp.full((B, J), -jnp.inf, jnp.float32)
        i_ref[...] = jnp.zeros((B, J), jnp.int32)

    iota = jax.lax.broadcasted_iota(jnp.int32, (B, VC), 1) + j * VC
    cms = []
    lis = []
    for jj in range(J):
        cmj = jnp.max(a[jj], axis=1, keepdims=True)
        lij = jnp.min(jnp.where(a[jj] == cmj, iota, V), axis=1, keepdims=True)
        cms.append(cmj)
        lis.append(lij)
    cm = jnp.concatenate(cms, axis=1)
    li = jnp.concatenate(lis, axis=1)
    upd = cm > m_ref[...]
    m_ref[...] = jnp.where(upd, cm, m_ref[...])
    i_ref[...] = jnp.where(upd, li, i_ref[...])
    best_ref[...] = i_ref[...]


@jax.jit
def kernel(action, explore_rate):
    a2 = action.reshape(B, J * V)
    e2 = explore_rate.reshape(B, J * V)
    jslice = lambda jj: pl.BlockSpec(
        (B, VC), lambda j, jj=jj: (0, jj * NCHUNK + j))
    logits, stddev, best2d = pl.pallas_call(
        _fused_body,
        grid=(NCHUNK,),
        in_specs=[jslice(jj) for jj in range(J)] * 2,
        out_specs=[
            pl.BlockSpec((B, VC), lambda j: (0, j)),
            pl.BlockSpec((B, VC), lambda j: (0, j)),
            pl.BlockSpec((B, J), lambda j: (0, 0)),
        ],
        out_shape=[
            jax.ShapeDtypeStruct((B, V), jnp.float32),
            jax.ShapeDtypeStruct((B, V), jnp.float32),
            jax.ShapeDtypeStruct((B, J), jnp.int32),
        ],
        scratch_shapes=[
            pltpu.VMEM((B, J), jnp.float32),
            pltpu.VMEM((B, J), jnp.int32),
        ],
        compiler_params=pltpu.CompilerParams(
            dimension_semantics=("arbitrary",),
        ),
    )(*([a2] * J), *([e2] * J))
    return logits, stddev, best2d[:, None, :]


# speedup vs baseline: 2.2637x; 2.2637x over previous
import jax
import jax.numpy as jnp
from jax.experimental import pallas as pl
from jax.experimental.pallas import tpu as pltpu

B, J, V = 32, 8, 32768
VC = 2048
NCHUNK = V // VC


def _fused_body(a_ref, e_ref, logits_ref, stddev_ref, best_ref,
                macc_ref, cidx_ref):
    j = pl.program_id(0)

    a = a_ref[...]
    e = e_ref[...]

    m = jnp.min(a, axis=1)
    iota_j = jax.lax.broadcasted_iota(jnp.int32, (B, J, VC), 1)
    jsel = jnp.min(jnp.where(a == m[:, None, :], iota_j, J), axis=1)
    s = jnp.sum(jnp.where(iota_j == jsel[:, None, :], e, 0.0), axis=1)
    logits_ref[...] = m
    stddev_ref[...] = s

    @pl.when(j == 0)
    def _():
        macc_ref[...] = a
        cidx_ref[...] = jnp.zeros((B, J, VC), jnp.int32)

    @pl.when(j > 0)
    def _():
        upd = a > macc_ref[...]
        macc_ref[...] = jnp.where(upd, a, macc_ref[...])
        cidx_ref[...] = jnp.where(upd, j, cidx_ref[...])

    @pl.when(j == NCHUNK - 1)
    def _():
        macc = macc_ref[...]
        cm = jnp.max(macc, axis=2)
        lane = jax.lax.broadcasted_iota(jnp.int32, (B, J, VC), 2)
        gidx = cidx_ref[...] * VC + lane
        best_ref[...] = jnp.min(
            jnp.where(macc == cm[:, :, None], gidx, V), axis=2)


@jax.jit
def kernel(action, explore_rate):
    logits, stddev, best2d = pl.pallas_call(
        _fused_body,
        grid=(NCHUNK,),
        in_specs=[
            pl.BlockSpec((B, J, VC), lambda j: (0, 0, j)),
            pl.BlockSpec((B, J, VC), lambda j: (0, 0, j)),
        ],
        out_specs=[
            pl.BlockSpec((B, VC), lambda j: (0, j)),
            pl.BlockSpec((B, VC), lambda j: (0, j)),
            pl.BlockSpec((B, J), lambda j: (0, 0)),
        ],
        out_shape=[
            jax.ShapeDtypeStruct((B, V), jnp.float32),
            jax.ShapeDtypeStruct((B, V), jnp.float32),
            jax.ShapeDtypeStruct((B, J), jnp.int32),
        ],
        scratch_shapes=[
            pltpu.VMEM((B, J, VC), jnp.float32),
            pltpu.VMEM((B, J, VC), jnp.int32),
        ],
        compiler_params=pltpu.CompilerParams(
            dimension_semantics=("arbitrary",),
        ),
    )(action, explore_rate)
    return logits, stddev, best2d[:, None, :]
